# Initial kernel scaffold; baseline (speedup 1.0000x reference)
import jax
import jax.numpy as jnp
from jax.experimental import pallas as pl


def kernel(drug_x, drug_edge_index, dis_x, dis_edge_index, W1d, b1d, W2d, b2d, W1s, b1s, W2s, b2s):
    raise NotImplementedError("write your pallas kernel here")


# speedup vs baseline: 3.7918x; 3.7918x over previous
import functools

import jax
import jax.numpy as jnp
from jax import lax
from jax.experimental import pallas as pl
from jax.experimental.pallas import tpu as pltpu
from jax.experimental.pallas import tpu_sc as plsc

N = 10000
F = 128
H = 128
E = 320000

NUM_TILES = 16
CHUNK = 128
CPT = -(-E // (NUM_TILES * CHUNK))
EPAD = CPT * NUM_TILES * CHUNK
PADROW = N
NACC = 10240
ZROWS_PER_TILE = NACC // NUM_TILES
OUT_PER_TILE = N // NUM_TILES
LANES = 16


def _matmul(x, w, rows_blk):
    def body(x_ref, w_ref, o_ref):
        o_ref[...] = jnp.dot(x_ref[...], w_ref[...],
                             preferred_element_type=jnp.float32)
    grid = (N // rows_blk,)
    return pl.pallas_call(
        body,
        grid=grid,
        in_specs=[
            pl.BlockSpec((rows_blk, F), lambda r: (r, 0)),
            pl.BlockSpec((F, H), lambda r: (0, 0)),
        ],
        out_specs=pl.BlockSpec((rows_blk, H), lambda r: (r, 0)),
        out_shape=jax.ShapeDtypeStruct((N, H), jnp.float32),
    )(x, w)


def _bias_relu_matmul(a, b, w, rows_blk):
    def body(a_ref, b_ref, w_ref, o_ref):
        h = jnp.maximum(a_ref[...] + b_ref[...], 0.0)
        o_ref[...] = jnp.dot(h, w_ref[...], preferred_element_type=jnp.float32)
    grid = (N // rows_blk,)
    return pl.pallas_call(
        body,
        grid=grid,
        in_specs=[
            pl.BlockSpec((rows_blk, H), lambda r: (r, 0)),
            pl.BlockSpec((1, H), lambda r: (0, 0)),
            pl.BlockSpec((H, H), lambda r: (0, 0)),
        ],
        out_specs=pl.BlockSpec((rows_blk, H), lambda r: (r, 0)),
        out_shape=jax.ShapeDtypeStruct((N, H), jnp.float32),
    )(a, b.reshape(1, H), w)


def _bias_add(a, b, rows_blk):
    def body(a_ref, b_ref, o_ref):
        o_ref[...] = a_ref[...] + b_ref[...]
    grid = (N // rows_blk,)
    return pl.pallas_call(
        body,
        grid=grid,
        in_specs=[
            pl.BlockSpec((rows_blk, H), lambda r: (r, 0)),
            pl.BlockSpec((1, H), lambda r: (0, 0)),
        ],
        out_specs=pl.BlockSpec((rows_blk, H), lambda r: (r, 0)),
        out_shape=jax.ShapeDtypeStruct((N, H), jnp.float32),
    )(a, b.reshape(1, H))


def _sc_scatter(md, ms, eid, eis):
    mesh = plsc.VectorSubcoreMesh(core_axis_name="c", subcore_axis_name="s")

    @functools.partial(
        pl.kernel,
        out_type=(
            jax.ShapeDtypeStruct((N, H), jnp.float32),
            jax.ShapeDtypeStruct((N, H), jnp.float32),
        ),
        mesh=mesh,
        scratch_types=[
            pltpu.VMEM_SHARED((NACC, H), jnp.float32),
            pltpu.VMEM((CHUNK, H), jnp.float32),
            pltpu.VMEM((CHUNK,), jnp.int32),
            pltpu.VMEM((CHUNK,), jnp.int32),
            pltpu.SemaphoreType.DMA,
        ],
    )
    def scatter_kernel(md_hbm, ms_hbm, eid_hbm, eis_hbm, outd_hbm, outs_hbm,
                       acc, rows, src_v, dst_v, sem):
        c = lax.axis_index("c")
        s = lax.axis_index("s")

        def zrow(i, _):
            def zlane(j, _):
                rows[i, pl.ds(j * LANES, LANES)] = jnp.zeros((LANES,), jnp.float32)
                return 0
            return lax.fori_loop(0, H // LANES, zlane, 0)
        lax.fori_loop(0, CHUNK, zrow, 0)

        zbase = s * ZROWS_PER_TILE
        def zcopy(k, _):
            pltpu.sync_copy(rows, acc.at[pl.ds(zbase + k * CHUNK, CHUNK)])
            return 0
        lax.fori_loop(0, ZROWS_PER_TILE // CHUNK, zcopy, 0)
        plsc.subcore_barrier()

        def run(m_hbm, ei_hbm, out_hbm):
            def step(j, _):
                ebase = (s * CPT + j) * CHUNK
                pltpu.sync_copy(ei_hbm.at[0, pl.ds(ebase, CHUNK)], src_v)
                pltpu.sync_copy(ei_hbm.at[1, pl.ds(ebase, CHUNK)], dst_v)
                pltpu.async_copy(m_hbm.at[src_v], rows, sem).wait()
                pltpu.sync_copy(rows, acc.at[dst_v], add=True)
                return 0
            lax.fori_loop(0, CPT, step, 0)
            plsc.subcore_barrier()
            obase = s * 640

            @pl.when(s < 15)
            def _():
                pltpu.sync_copy(acc.at[pl.ds(obase, 640)],
                                out_hbm.at[pl.ds(obase, 640)])

            @pl.when(s == 15)
            def _():
                pltpu.sync_copy(acc.at[pl.ds(9600, 400)],
                                out_hbm.at[pl.ds(9600, 400)])

        @pl.when(c == 0)
        def _():
            run(md_hbm, eid_hbm, outd_hbm)

        @pl.when(c == 1)
        def _():
            run(ms_hbm, eis_hbm, outs_hbm)

    return scatter_kernel(md, ms, eid, eis)


def _pad_edges(ei):
    pad = EPAD - E
    pad_cols = jnp.concatenate([
        jnp.zeros((1, pad), jnp.int32),
        jnp.full((1, pad), PADROW, jnp.int32),
    ], axis=0)
    return jnp.concatenate([ei, pad_cols], axis=1)


def kernel(drug_x, drug_edge_index, dis_x, dis_edge_index,
           W1d, b1d, W2d, b2d, W1s, b1s, W2s, b2s):
    eid = _pad_edges(drug_edge_index)
    eis = _pad_edges(dis_edge_index)

    rows_blk = 1000
    m1d = _matmul(drug_x, W1d, rows_blk)
    m1s = _matmul(dis_x, W1s, rows_blk)
    agg1d, agg1s = _sc_scatter(m1d, m1s, eid, eis)
    m2d = _bias_relu_matmul(agg1d, b1d, W2d, rows_blk)
    m2s = _bias_relu_matmul(agg1s, b1s, W2s, rows_blk)
    agg2d, agg2s = _sc_scatter(m2d, m2s, eid, eis)
    emb1 = _bias_add(agg2d, b2d, rows_blk)
    emb2 = _bias_add(agg2s, b2s, rows_blk)
    return (emb1, emb2)
